# baseline (speedup 1.0000x reference)
import jax
import jax.numpy as jnp
from jax.experimental import pallas as pl
from jax.experimental.pallas import tpu as pltpu

TOPK = 13
EPS = 1e-09


def _rot_minmax(cx, cy, w, h, r):
    cos_r = jnp.cos(r)
    sin_r = jnp.sin(r)
    dx = w / 2.0 * cos_r
    dy = h / 2.0 * sin_r
    xs = (cx - dx, cx + dx, cx + dx, cx - dx)
    ys = (cy - dy, cy - dy, cy + dy, cy + dy)
    x_rot = [cx + (xc - cx) * cos_r - (yc - cy) * sin_r for xc, yc in zip(xs, ys)]
    y_rot = [cy + (xc - cx) * sin_r + (yc - cy) * cos_r for xc, yc in zip(xs, ys)]
    min_x = jnp.minimum(jnp.minimum(x_rot[0], x_rot[1]), jnp.minimum(x_rot[2], x_rot[3]))
    max_x = jnp.maximum(jnp.maximum(x_rot[0], x_rot[1]), jnp.maximum(x_rot[2], x_rot[3]))
    min_y = jnp.minimum(jnp.minimum(y_rot[0], y_rot[1]), jnp.minimum(y_rot[2], y_rot[3]))
    max_y = jnp.maximum(jnp.maximum(y_rot[0], y_rot[1]), jnp.maximum(y_rot[2], y_rot[3]))
    return min_x, max_x, min_y, max_y


def _assign_body(bg_ref, ps_ref, prt_ref, apt_ref, gl_ref, gb_ref, gc_ref,
                 lab_ref, rb_ref, sc_ref, agi_ref, crowd_ref, x_ref):
    n = gb_ref.shape[1]
    L = prt_ref.shape[2]
    C = ps_ref.shape[2]
    f32 = jnp.float32

    ps = ps_ref[0]
    prt = prt_ref[0]
    apt = apt_ref[...]
    gl = gl_ref[0]
    gb = gb_ref[0]
    gc = gc_ref[0]
    bg = bg_ref[0]

    pcx, pcy, pw, ph, prr = (prt[i:i + 1, :] for i in range(5))
    p_minx, p_maxx, p_miny, p_maxy = _rot_minmax(pcx, pcy, pw, ph, prr)
    p_area = pw * ph

    gcx, gcy, gw, gh, grr = (gb[:, i:i + 1] for i in range(5))
    g_minx, g_maxx, g_miny, g_maxy = _rot_minmax(gcx, gcy, gw, gh, grr)
    g_area = gw * gh

    iw = jnp.clip(jnp.minimum(g_maxx, p_maxx) - jnp.maximum(g_minx, p_minx), 0.0, None)
    ih = jnp.clip(jnp.minimum(g_maxy, p_maxy) - jnp.maximum(g_miny, p_miny), 0.0, None)
    inter = iw * ih
    union = g_area + p_area - inter
    ious = jnp.clip(inter / (union + EPS), 0.0, 1.0)

    cls_iota = jax.lax.broadcasted_iota(jnp.int32, (n, C), 1)
    onehot_lbl = (cls_iota == gl).astype(f32)
    bbox_cls = jax.lax.dot_general(
        onehot_lbl, ps, (((1,), (1,)), ((), ())),
        preferred_element_type=f32)

    iou2 = ious * ious
    iou3 = iou2 * ious
    metrics = bbox_cls * (iou3 * iou3)

    px = apt[0:1, :]
    py = apt[1:2, :]
    cos_g = jnp.cos(grr)
    sin_g = jnp.sin(grr)
    dxp = px - gcx
    dyp = py - gcy
    lx = dxp * cos_g + dyp * sin_g
    ly = -dxp * sin_g + dyp * cos_g
    in_b = (jnp.abs(lx) <= gw / 2.0) & (jnp.abs(ly) <= gh / 2.0)

    iota_lf = jax.lax.broadcasted_iota(jnp.int32, (n, L), 1).astype(f32)
    Lf = float(L)
    DETIE = 2.0 ** -100
    zmap = iota_lf * (-DETIE) - DETIE
    xt = jnp.where(in_b & (metrics > 0.0), metrics, zmap)
    for _ in range(TOPK):
        v = jnp.max(xt, axis=1, keepdims=True)
        xt = jnp.where(xt == v, -2.0, xt)
    x_ref[...] = xt
    n_marked = jnp.sum(jnp.where(xt == -2.0, 1.0, 0.0))

    @pl.when(n_marked != 13.0 * n)
    def _exact_tie_fallback():
        xs = jnp.where(in_b & (metrics > 0.0), metrics, zmap)
        for _ in range(TOPK):
            v = jnp.max(xs, axis=1, keepdims=True)
            cand = jnp.where(xs == v, iota_lf, Lf)
            j = jnp.min(cand, axis=1, keepdims=True)
            xs = jnp.where(iota_lf == j, -2.0, xs)
        x_ref[...] = xs

    mask_positive = jnp.where((x_ref[...] == -2.0) & in_b, 1.0, 0.0)
    mp_sum = jnp.sum(mask_positive, axis=0, keepdims=True)

    iou_colmax = jnp.max(ious, axis=0, keepdims=True)
    iota_nf = jax.lax.broadcasted_iota(jnp.int32, (n, L), 0).astype(f32)
    nf = float(n)
    first_max_f = jnp.min(jnp.where(ious == iou_colmax, iota_nf, nf),
                          axis=0, keepdims=True)
    is_max_iou = (iota_nf == first_max_f).astype(f32)
    multi = mp_sum > 1.0
    single_idx = jnp.sum(iota_nf * mask_positive, axis=0, keepdims=True)
    mask_positive = jnp.where(multi, is_max_iou, mask_positive)
    mp_sum = jnp.where(multi, 1.0, mp_sum)

    agi_f = jnp.where(multi, first_max_f, single_idx)
    oh_agi = (iota_nf == agi_f).astype(f32)

    combo = (gl * 2 + gc).astype(f32)
    combo_sum = jnp.sum(oh_agi * combo, axis=0, keepdims=True)
    combo_i = combo_sum.astype(jnp.int32)
    has_pos = mp_sum > 0.0
    lab = jnp.where(has_pos, combo_i >> 1, bg)
    crowd_i = combo_i & 1

    rbox = jax.lax.dot_general(
        oh_agi, gb, (((0,), (0,)), ((), ())),
        preferred_element_type=f32)

    am_full = metrics * mask_positive
    max_m = jnp.max(am_full, axis=1, keepdims=True)
    max_i = jnp.max(ious * mask_positive, axis=1, keepdims=True)
    rowscale = max_i / (max_m + EPS)
    am_final = jnp.max(am_full * rowscale, axis=0, keepdims=True)

    w_row = am_final * (crowd_i == 0).astype(f32) * has_pos.astype(f32)
    scores = jax.lax.dot_general(
        oh_agi * w_row, onehot_lbl, (((0,), (0,)), ((), ())),
        preferred_element_type=f32)

    lab_ref[0] = lab
    agi_ref[0] = agi_f.astype(jnp.int32)
    crowd_ref[0] = crowd_i
    rb_ref[0] = rbox
    sc_ref[0] = scores


def _run(pred_scores, pred_rboxes, anchor_points, gt_labels, gt_bboxes,
         gt_crowd, bg_arr):
    B, L, C = pred_scores.shape
    n = gt_bboxes.shape[1]
    f32 = jnp.float32

    prt = jnp.transpose(pred_rboxes, (0, 2, 1))
    apt = jnp.transpose(anchor_points, (1, 0))

    out_shapes = (
        jax.ShapeDtypeStruct((B, 1, L), jnp.int32),
        jax.ShapeDtypeStruct((B, L, 5), f32),
        jax.ShapeDtypeStruct((B, L, C), f32),
        jax.ShapeDtypeStruct((B, 1, L), jnp.int32),
        jax.ShapeDtypeStruct((B, 1, L), jnp.int32),
    )
    grid = (B,)
    in_specs = [
        pl.BlockSpec(memory_space=pltpu.SMEM),
        pl.BlockSpec((1, L, C), lambda b: (b, 0, 0)),
        pl.BlockSpec((1, 5, L), lambda b: (b, 0, 0)),
        pl.BlockSpec((2, L), lambda b: (0, 0)),
        pl.BlockSpec((1, n, 1), lambda b: (b, 0, 0)),
        pl.BlockSpec((1, n, 5), lambda b: (b, 0, 0)),
        pl.BlockSpec((1, n, 1), lambda b: (b, 0, 0)),
    ]
    out_specs = (
        pl.BlockSpec((1, 1, L), lambda b: (b, 0, 0)),
        pl.BlockSpec((1, L, 5), lambda b: (b, 0, 0)),
        pl.BlockSpec((1, L, C), lambda b: (b, 0, 0)),
        pl.BlockSpec((1, 1, L), lambda b: (b, 0, 0)),
        pl.BlockSpec((1, 1, L), lambda b: (b, 0, 0)),
    )
    lab, rb, sc, agi, crowd = pl.pallas_call(
        _assign_body,
        grid=grid,
        in_specs=in_specs,
        out_specs=out_specs,
        out_shape=out_shapes,
        scratch_shapes=[pltpu.VMEM((n, L), f32)],
    )(bg_arr, pred_scores, prt, apt, gt_labels, gt_bboxes, gt_crowd)
    return (lab.reshape(B, L), rb, sc, agi.reshape(B, L), crowd.reshape(B, L))


def kernel(pred_scores, pred_rboxes, anchor_points, gt_labels, gt_bboxes,
           gt_poses, gt_crowd, pad_gt_mask, bg_index):
    del gt_poses, pad_gt_mask
    bg_arr = jnp.asarray(bg_index, jnp.int32).reshape(1)
    return _run(pred_scores, pred_rboxes, anchor_points, gt_labels,
                gt_bboxes, gt_crowd, bg_arr)


# speedup vs baseline: 1.0245x; 1.0245x over previous
import jax
import jax.numpy as jnp
from jax.experimental import pallas as pl
from jax.experimental.pallas import tpu as pltpu

TOPK = 13
EPS = 1e-09


def _rot_minmax(cx, cy, w, h, r):
    cos_r = jnp.cos(r)
    sin_r = jnp.sin(r)
    dx = w / 2.0 * cos_r
    dy = h / 2.0 * sin_r
    xs = (cx - dx, cx + dx, cx + dx, cx - dx)
    ys = (cy - dy, cy - dy, cy + dy, cy + dy)
    x_rot = [cx + (xc - cx) * cos_r - (yc - cy) * sin_r for xc, yc in zip(xs, ys)]
    y_rot = [cy + (xc - cx) * sin_r + (yc - cy) * cos_r for xc, yc in zip(xs, ys)]
    min_x = jnp.minimum(jnp.minimum(x_rot[0], x_rot[1]), jnp.minimum(x_rot[2], x_rot[3]))
    max_x = jnp.maximum(jnp.maximum(x_rot[0], x_rot[1]), jnp.maximum(x_rot[2], x_rot[3]))
    min_y = jnp.minimum(jnp.minimum(y_rot[0], y_rot[1]), jnp.minimum(y_rot[2], y_rot[3]))
    max_y = jnp.maximum(jnp.maximum(y_rot[0], y_rot[1]), jnp.maximum(y_rot[2], y_rot[3]))
    return min_x, max_x, min_y, max_y


def _assign_body(bg_ref, ps_ref, prt_ref, apt_ref, gl_ref, gb_ref, gc_ref,
                 lab_ref, rb_ref, sc_ref, agi_ref, crowd_ref, x_ref):
    n = gb_ref.shape[1]
    L = prt_ref.shape[2]
    C = ps_ref.shape[2]
    f32 = jnp.float32

    ps = ps_ref[0]
    prt = prt_ref[0]
    apt = apt_ref[...]
    gl = gl_ref[0]
    gb = gb_ref[0]
    gc = gc_ref[0]
    bg = bg_ref[0]

    pcx, pcy, pw, ph, prr = (prt[i:i + 1, :] for i in range(5))
    p_minx, p_maxx, p_miny, p_maxy = _rot_minmax(pcx, pcy, pw, ph, prr)
    p_area = pw * ph

    gcx, gcy, gw, gh, grr = (gb[:, i:i + 1] for i in range(5))
    g_minx, g_maxx, g_miny, g_maxy = _rot_minmax(gcx, gcy, gw, gh, grr)
    g_area = gw * gh

    iw = jnp.clip(jnp.minimum(g_maxx, p_maxx) - jnp.maximum(g_minx, p_minx), 0.0, None)
    ih = jnp.clip(jnp.minimum(g_maxy, p_maxy) - jnp.maximum(g_miny, p_miny), 0.0, None)
    inter = iw * ih
    union = g_area + p_area - inter
    ious = jnp.clip(inter / (union + EPS), 0.0, 1.0)

    cls_iota = jax.lax.broadcasted_iota(jnp.int32, (n, C), 1)
    onehot_lbl = (cls_iota == gl).astype(f32)
    bbox_cls = jax.lax.dot_general(
        onehot_lbl, ps, (((1,), (1,)), ((), ())),
        preferred_element_type=f32)

    iou2 = ious * ious
    iou3 = iou2 * ious
    metrics = bbox_cls * (iou3 * iou3)

    px = apt[0:1, :]
    py = apt[1:2, :]
    cos_g = jnp.cos(grr)
    sin_g = jnp.sin(grr)
    dxp = px - gcx
    dyp = py - gcy
    lx = dxp * cos_g + dyp * sin_g
    ly = -dxp * sin_g + dyp * cos_g
    in_gts = ((jnp.abs(lx) <= gw / 2.0) & (jnp.abs(ly) <= gh / 2.0)).astype(f32)

    iota_lf = jax.lax.broadcasted_iota(jnp.int32, (n, L), 1).astype(f32)
    Lf = float(L)
    DETIE = 2.0 ** -100
    x = metrics * in_gts
    zmap = iota_lf * (-DETIE) - DETIE
    xt = jnp.where(x > 0.0, x, zmap)
    for _ in range(TOPK):
        v = jnp.max(xt, axis=1, keepdims=True)
        xt = jnp.where(xt == v, -2.0, xt)
    x_ref[...] = xt
    n_marked = jnp.sum(jnp.where(xt == -2.0, 1.0, 0.0))

    @pl.when(n_marked != 13.0 * n)
    def _exact_tie_fallback():
        xs = jnp.where(x > 0.0, x, zmap)
        for _ in range(TOPK):
            v = jnp.max(xs, axis=1, keepdims=True)
            cand = jnp.where(xs == v, iota_lf, Lf)
            j = jnp.min(cand, axis=1, keepdims=True)
            xs = jnp.where(iota_lf == j, -2.0, xs)
        x_ref[...] = xs

    mask_positive = jnp.where(x_ref[...] == -2.0, in_gts, 0.0)
    mp_sum = jnp.sum(mask_positive, axis=0, keepdims=True)

    iou_colmax = jnp.max(ious, axis=0, keepdims=True)
    iota_nf = jax.lax.broadcasted_iota(jnp.int32, (n, L), 0).astype(f32)
    nf = float(n)
    first_max_f = jnp.min(jnp.where(ious == iou_colmax, iota_nf, nf),
                          axis=0, keepdims=True)
    is_max_iou = (iota_nf == first_max_f).astype(f32)
    multi = mp_sum > 1.0
    single_idx = jnp.sum(iota_nf * mask_positive, axis=0, keepdims=True)
    mask_positive = jnp.where(multi, is_max_iou, mask_positive)
    mp_sum = jnp.where(multi, 1.0, mp_sum)

    agi_f = jnp.where(multi, first_max_f, single_idx)
    oh_agi = (iota_nf == agi_f).astype(f32)

    combo = (gl * 2 + gc).astype(f32)
    combo_sum = jnp.sum(oh_agi * combo, axis=0, keepdims=True)
    combo_i = combo_sum.astype(jnp.int32)
    has_pos = mp_sum > 0.0
    lab = jnp.where(has_pos, combo_i >> 1, bg)
    crowd_i = combo_i & 1

    rbox = jax.lax.dot_general(
        oh_agi, gb, (((0,), (0,)), ((), ())),
        preferred_element_type=f32)

    am_full = metrics * mask_positive
    max_m = jnp.max(am_full, axis=1, keepdims=True)
    max_i = jnp.max(ious * mask_positive, axis=1, keepdims=True)
    rowscale = max_i / (max_m + EPS)
    am_final = jnp.max(am_full * rowscale, axis=0, keepdims=True)

    w_row = am_final * (crowd_i == 0).astype(f32) * has_pos.astype(f32)
    scores = jax.lax.dot_general(
        oh_agi * w_row, onehot_lbl, (((0,), (0,)), ((), ())),
        preferred_element_type=f32)

    lab_ref[0] = lab
    agi_ref[0] = agi_f.astype(jnp.int32)
    crowd_ref[0] = crowd_i
    rb_ref[0] = rbox
    sc_ref[0] = scores


def _run(pred_scores, pred_rboxes, anchor_points, gt_labels, gt_bboxes,
         gt_crowd, bg_arr):
    B, L, C = pred_scores.shape
    n = gt_bboxes.shape[1]
    f32 = jnp.float32

    prt = jnp.transpose(pred_rboxes, (0, 2, 1))
    apt = jnp.transpose(anchor_points, (1, 0))

    out_shapes = (
        jax.ShapeDtypeStruct((B, 1, L), jnp.int32),
        jax.ShapeDtypeStruct((B, L, 5), f32),
        jax.ShapeDtypeStruct((B, L, C), f32),
        jax.ShapeDtypeStruct((B, 1, L), jnp.int32),
        jax.ShapeDtypeStruct((B, 1, L), jnp.int32),
    )
    grid = (B,)
    in_specs = [
        pl.BlockSpec(memory_space=pltpu.SMEM),
        pl.BlockSpec((1, L, C), lambda b: (b, 0, 0)),
        pl.BlockSpec((1, 5, L), lambda b: (b, 0, 0)),
        pl.BlockSpec((2, L), lambda b: (0, 0)),
        pl.BlockSpec((1, n, 1), lambda b: (b, 0, 0)),
        pl.BlockSpec((1, n, 5), lambda b: (b, 0, 0)),
        pl.BlockSpec((1, n, 1), lambda b: (b, 0, 0)),
    ]
    out_specs = (
        pl.BlockSpec((1, 1, L), lambda b: (b, 0, 0)),
        pl.BlockSpec((1, L, 5), lambda b: (b, 0, 0)),
        pl.BlockSpec((1, L, C), lambda b: (b, 0, 0)),
        pl.BlockSpec((1, 1, L), lambda b: (b, 0, 0)),
        pl.BlockSpec((1, 1, L), lambda b: (b, 0, 0)),
    )
    lab, rb, sc, agi, crowd = pl.pallas_call(
        _assign_body,
        grid=grid,
        in_specs=in_specs,
        out_specs=out_specs,
        out_shape=out_shapes,
        scratch_shapes=[pltpu.VMEM((n, L), f32)],
    )(bg_arr, pred_scores, prt, apt, gt_labels, gt_bboxes, gt_crowd)
    return (lab.reshape(B, L), rb, sc, agi.reshape(B, L), crowd.reshape(B, L))


def kernel(pred_scores, pred_rboxes, anchor_points, gt_labels, gt_bboxes,
           gt_poses, gt_crowd, pad_gt_mask, bg_index):
    del gt_poses, pad_gt_mask
    bg_arr = jnp.asarray(bg_index, jnp.int32).reshape(1)
    return _run(pred_scores, pred_rboxes, anchor_points, gt_labels,
                gt_bboxes, gt_crowd, bg_arr)
